# Initial kernel scaffold; baseline (speedup 1.0000x reference)
import jax
import jax.numpy as jnp
from jax.experimental import pallas as pl


def kernel(x, edge_index, edge_attr, edge_sh, Wt_r, Wt_i, M1, b1, M2, b2, A, Bsh):
    raise NotImplementedError("write your pallas kernel here")


# speedup vs baseline: 2.9186x; 2.9186x over previous
import functools

import jax
import jax.numpy as jnp
from jax import lax
from jax.experimental import pallas as pl
from jax.experimental.pallas import tpu as pltpu
from jax.experimental.pallas import tpu_sc as plsc

NS = 16
PAD = 32
CNT = 28

SC_CORES = 2
SC_SUBCORES = 16
NW = SC_CORES * SC_SUBCORES


def _node_body(x_ref, wtr_ref, a_ref, o_ref):
    xb = x_ref[...]
    wr = wtr_ref[..., 0] + wtr_ref[..., 1]
    a = a_ref[...]
    xs2 = jnp.dot(xb[:, :NS], wr, preferred_element_type=jnp.float32)
    y = (jnp.dot(xs2, a[:NS, :], preferred_element_type=jnp.float32)
         + jnp.dot(xb[:, NS:], a[NS:, :], preferred_element_type=jnp.float32))
    bn = y.shape[0]
    o_ref[...] = jnp.concatenate(
        [y, jnp.ones((bn, 1), jnp.float32), jnp.zeros((bn, PAD - CNT - 1), jnp.float32)],
        axis=1)


def _edge_body(ea_ref, es_ref, m1_ref, b1_ref, m2_ref, b2_ref, bsh_ref, o_ref):
    h = jnp.maximum(jnp.dot(ea_ref[...], m1_ref[...],
                            preferred_element_type=jnp.float32) + b1_ref[...], 0.0)
    ew = jnp.dot(h, m2_ref[...], preferred_element_type=jnp.float32) + b2_ref[...]
    shp = jnp.dot(es_ref[...], bsh_ref[...], preferred_element_type=jnp.float32)
    d = ew * shp
    be = d.shape[0]
    o_ref[...] = jnp.concatenate(
        [d, jnp.ones((be, 1), jnp.float32), jnp.zeros((be, PAD - CNT - 1), jnp.float32)],
        axis=1)


def _combine_body(p_ref, o_ref):
    pb = p_ref[0] + p_ref[1]
    cnt = jnp.maximum(pb[:, CNT:CNT + 1], 1.0)
    o_ref[...] = pb[:, :CNT] / cnt


def _sc_scatter(n_nodes, n_edges):
    ew_per_tile = n_edges // NW
    K = 128
    nch = ew_per_tile // K
    tail = ew_per_tile - nch * K
    rows_per_sub = n_nodes // SC_SUBCORES
    ZR = 125
    nz = rows_per_sub // ZR

    mesh = plsc.VectorSubcoreMesh(core_axis_name="c", subcore_axis_name="s",
                                  num_cores=SC_CORES, num_subcores=SC_SUBCORES)

    scratch = [
        pltpu.VMEM((K,), jnp.int32),
        pltpu.VMEM((K,), jnp.int32),
        pltpu.VMEM((K, PAD), jnp.float32),
        pltpu.VMEM((K, PAD), jnp.float32),
        pltpu.VMEM((ZR, PAD), jnp.float32),
        pltpu.VMEM_SHARED((n_nodes, PAD), jnp.float32),
        pltpu.SemaphoreType.DMA,
    ]
    if tail:
        scratch += [
            pltpu.VMEM((tail,), jnp.int32),
            pltpu.VMEM((tail,), jnp.int32),
            pltpu.VMEM((tail, PAD), jnp.float32),
            pltpu.VMEM((tail, PAD), jnp.float32),
        ]

    @functools.partial(
        pl.kernel,
        out_type=jax.ShapeDtypeStruct((SC_CORES, n_nodes, PAD), jnp.float32),
        mesh=mesh,
        scratch_types=scratch,
        compiler_params=pltpu.CompilerParams(use_tc_tiling_on_sc=False),
    )
    def run(src_hbm, dst_hbm, y_hbm, d_hbm, out_hbm, si, di, rows, dv, zbuf,
            acc, sem, *tails):
        c = lax.axis_index("c")
        s = lax.axis_index("s")
        wid = c * SC_SUBCORES + s
        zero16 = jnp.zeros((16,), jnp.float32)

        def zb(i, carry):
            zbuf[i, pl.ds(0, 16)] = zero16
            zbuf[i, pl.ds(16, 16)] = zero16
            return carry
        lax.fori_loop(0, ZR, zb, 0)

        def zc(k, carry):
            pltpu.sync_copy(zbuf, acc.at[pl.ds(s * rows_per_sub + k * ZR, ZR)])
            return carry
        lax.fori_loop(0, nz, zc, 0)
        plsc.subcore_barrier()

        base0 = wid * ew_per_tile

        def do_chunk(base, si_, di_, rows_, dv_, kk):
            pltpu.sync_copy(src_hbm.at[pl.ds(base, kk)], si_)
            pltpu.sync_copy(dst_hbm.at[pl.ds(base, kk)], di_)
            pltpu.sync_copy(d_hbm.at[pl.ds(base, kk)], dv_)
            pltpu.async_copy(y_hbm.at[si_], rows_, sem).wait()

            def mul(i, carry):
                rows_[i, pl.ds(0, 16)] = rows_[i, pl.ds(0, 16)] * dv_[i, pl.ds(0, 16)]
                rows_[i, pl.ds(16, 16)] = rows_[i, pl.ds(16, 16)] * dv_[i, pl.ds(16, 16)]
                return carry
            lax.fori_loop(0, kk, mul, 0)
            pltpu.sync_copy(rows_, acc.at[di_], add=True)

        def chunk(ch, carry):
            do_chunk(base0 + ch * K, si, di, rows, dv, K)
            return carry
        lax.fori_loop(0, nch, chunk, 0)
        if tail:
            sit, dit, rowst, dvt = tails
            do_chunk(base0 + nch * K, sit, dit, rowst, dvt, tail)

        plsc.subcore_barrier()
        pltpu.sync_copy(acc.at[pl.ds(s * rows_per_sub, rows_per_sub)],
                        out_hbm.at[c, pl.ds(s * rows_per_sub, rows_per_sub)])

    return run


def kernel(x, edge_index, edge_attr, edge_sh, Wt_r, Wt_i, M1, b1, M2, b2, A, Bsh):
    n_nodes = x.shape[1]
    n_edges = edge_index.shape[1]
    src = edge_index[0]
    dst = edge_index[1]

    BN = 2000
    y_pad = pl.pallas_call(
        _node_body,
        grid=(n_nodes // BN,),
        in_specs=[
            pl.BlockSpec((BN, x.shape[2]), lambda i: (i, 0)),
            pl.BlockSpec(Wt_r.shape, lambda i: (0, 0, 0)),
            pl.BlockSpec(A.shape, lambda i: (0, 0)),
        ],
        out_specs=pl.BlockSpec((BN, PAD), lambda i: (i, 0)),
        out_shape=jax.ShapeDtypeStruct((n_nodes, PAD), jnp.float32),
    )(x[0], Wt_r, A)

    BE = 8000
    d_pad = pl.pallas_call(
        _edge_body,
        grid=(n_edges // BE,),
        in_specs=[
            pl.BlockSpec((BE, edge_attr.shape[1]), lambda i: (i, 0)),
            pl.BlockSpec((BE, edge_sh.shape[1]), lambda i: (i, 0)),
            pl.BlockSpec(M1.shape, lambda i: (0, 0)),
            pl.BlockSpec((1, b1.shape[0]), lambda i: (0, 0)),
            pl.BlockSpec(M2.shape, lambda i: (0, 0)),
            pl.BlockSpec((1, b2.shape[0]), lambda i: (0, 0)),
            pl.BlockSpec(Bsh.shape, lambda i: (0, 0)),
        ],
        out_specs=pl.BlockSpec((BE, PAD), lambda i: (i, 0)),
        out_shape=jax.ShapeDtypeStruct((n_edges, PAD), jnp.float32),
    )(edge_attr, edge_sh, M1, b1.reshape(1, -1), M2, b2.reshape(1, -1), Bsh)

    partials = _sc_scatter(n_nodes, n_edges)(src, dst, y_pad, d_pad)

    out = pl.pallas_call(
        _combine_body,
        grid=(n_nodes // BN,),
        in_specs=[pl.BlockSpec((SC_CORES, BN, PAD), lambda i: (0, i, 0))],
        out_specs=pl.BlockSpec((BN, CNT), lambda i: (i, 0)),
        out_shape=jax.ShapeDtypeStruct((n_nodes, CNT), jnp.float32),
    )(partials)

    return out[None]
